# baseline (speedup 1.0000x reference)
import functools

import jax
import jax.numpy as jnp
from jax import lax
from jax.experimental import pallas as pl
from jax.experimental.pallas import tpu as pltpu
from jax.experimental.pallas import tpu_sc as plsc

_N = 10000
_NG = 64
_NC = 10
_NW = 32
_CH = 128
_NACC = 10240
_RPS = _NACC // 16


def _leaky(z):
  return jnp.maximum(z, 0.0) + 0.2 * jnp.minimum(z, 0.0)


def _splat8(vec, eq8):
  return jnp.full((16,), jnp.sum(jnp.where(eq8, vec, 0.0), axis=0))


def _k1_body(x_ref, w_ref, as_ref, ad_ref, t1_ref, d1_ref):
  h = jnp.dot(x_ref[...], w_ref[...], preferred_element_type=jnp.float32)
  asrc = jnp.dot(h, as_ref[...], preferred_element_type=jnp.float32)
  adst = jnp.dot(h, ad_ref[...], preferred_element_type=jnp.float32)
  t1_ref[...] = jnp.concatenate([h, asrc, asrc], axis=1)
  d1_ref[...] = jnp.concatenate([adst, adst], axis=1)


def _k1(x, W1, A1s, A1d):
  br = 1000
  grid = _N // br
  return pl.pallas_call(
      _k1_body,
      grid=(grid,),
      in_specs=[
          pl.BlockSpec((br, 128), lambda i: (i, 0)),
          pl.BlockSpec((128, 64), lambda i: (0, 0)),
          pl.BlockSpec((64, 8), lambda i: (0, 0)),
          pl.BlockSpec((64, 8), lambda i: (0, 0)),
      ],
      out_specs=[
          pl.BlockSpec((br, 80), lambda i: (i, 0)),
          pl.BlockSpec((br, 16), lambda i: (i, 0)),
      ],
      out_shape=[
          jax.ShapeDtypeStruct((_N, 80), jnp.float32),
          jax.ShapeDtypeStruct((_N, 16), jnp.float32),
      ],
  )(x, W1, A1s, A1d)


def _edge_pass(width, nchunks):
  mesh = plsc.VectorSubcoreMesh(core_axis_name="c", subcore_axis_name="s")
  assert nchunks % 2 == 0
  k2 = nchunks // 2

  def body(t_hbm, d_hbm, idx_hbm, acc_hbm,
           idxall, sbufA, sbufB, dbufA, dbufB, mbufA, mbufB, accs,
           gsemA, gsemB, ssemA, ssemB):
    c = lax.axis_index("c")
    s = lax.axis_index("s")
    wid = c * 16 + s
    lane = lax.iota(jnp.int32, 16)
    lt8 = lane < 8
    eq8 = lane == 8
    zero16 = (lane * 0).astype(jnp.float32)

    pltpu.sync_copy(idx_hbm.at[wid], idxall)

    def zrow(r, _):
      for j in range(width // 16):
        mbufA[r, j * 16:(j + 1) * 16] = zero16
      return 0
    lax.fori_loop(0, _CH, zrow, 0)
    for t in range(_RPS // _CH):
      pltpu.sync_copy(mbufA, accs.at[pl.ds(s * _RPS + t * _CH, _CH)])
    plsc.subcore_barrier()

    def gathers(k, sbuf, dbuf, gsem):
      pltpu.async_copy(t_hbm.at[idxall.at[k, 0]], sbuf, gsem)
      pltpu.async_copy(d_hbm.at[idxall.at[k, 1]], dbuf, gsem)

    def wait_gathers(k, sbuf, dbuf, gsem):
      pltpu.make_async_copy(t_hbm.at[idxall.at[k, 0]], sbuf, gsem).wait()
      pltpu.make_async_copy(d_hbm.at[idxall.at[k, 1]], dbuf, gsem).wait()

    def wait_scatter(k, mbuf, ssem):
      pltpu.make_async_copy(mbuf, accs.at[idxall.at[k, 1]], ssem).wait()

    if width == 80:
      def one_row(r, sbuf, dbuf, mbuf):
        asr = sbuf[r, 64:80]
        ads = dbuf[r, 0:16]
        e16 = jnp.exp(_leaky(asr + ads))
        for j in range(4):
          mbuf[r, j * 16:(j + 1) * 16] = e16 * sbuf[r, j * 16:(j + 1) * 16]
        mbuf[r, 64:80] = jnp.where(lt8, e16, 0.0)
    else:
      def one_row(r, sbuf, dbuf, mbuf):
        v = sbuf[r, 0:16]
        w = dbuf[r, 0:16]
        a = v[8] + w[8]
        e16 = jnp.exp(_leaky(jnp.full((16,), a)))
        mbuf[r, 0:16] = jnp.where(lt8, e16 * v, jnp.where(eq8, e16, 0.0))

    def compute(sbuf, dbuf, mbuf):
      @plsc.parallel_loop(0, _CH, 1, unroll=4)
      def _rows(r):
        one_row(r, sbuf, dbuf, mbuf)

    gathers(0, sbufA, dbufA, gsemA)

    def pair(i, _):
      ka = 2 * i
      kb = 2 * i + 1
      gathers(kb, sbufB, dbufB, gsemB)
      wait_gathers(ka, sbufA, dbufA, gsemA)
      pl.when(i > 0)(lambda: wait_scatter(ka, mbufA, ssemA))
      compute(sbufA, dbufA, mbufA)
      pltpu.async_copy(mbufA, accs.at[idxall.at[ka, 1]], ssemA, add=True)
      pl.when(i < k2 - 1)(lambda: gathers(ka + 2, sbufA, dbufA, gsemA))
      wait_gathers(kb, sbufB, dbufB, gsemB)
      pl.when(i > 0)(lambda: wait_scatter(kb, mbufB, ssemB))
      compute(sbufB, dbufB, mbufB)
      pltpu.async_copy(mbufB, accs.at[idxall.at[kb, 1]], ssemB, add=True)
      return 0

    lax.fori_loop(0, k2, pair, 0)
    wait_scatter(nchunks - 2, mbufA, ssemA)
    wait_scatter(nchunks - 1, mbufB, ssemB)
    plsc.subcore_barrier()
    for t in range(_RPS // _CH):
      r0 = s * _RPS + t * _CH
      pltpu.sync_copy(accs.at[pl.ds(r0, _CH)], acc_hbm.at[c, pl.ds(r0, _CH)])

  return pl.kernel(
      body,
      out_type=jax.ShapeDtypeStruct((2, _NACC, width), jnp.float32),
      mesh=mesh,
      scratch_types=[
          pltpu.VMEM((nchunks, 2, _CH), jnp.int32),
          pltpu.VMEM((_CH, width), jnp.float32),
          pltpu.VMEM((_CH, width), jnp.float32),
          pltpu.VMEM((_CH, 16), jnp.float32),
          pltpu.VMEM((_CH, 16), jnp.float32),
          pltpu.VMEM((_CH, width), jnp.float32),
          pltpu.VMEM((_CH, width), jnp.float32),
          pltpu.VMEM_SHARED((_NACC, width), jnp.float32),
          pltpu.SemaphoreType.DMA,
          pltpu.SemaphoreType.DMA,
          pltpu.SemaphoreType.DMA,
          pltpu.SemaphoreType.DMA,
      ],
      compiler_params=pltpu.CompilerParams(use_tc_tiling_on_sc=False,
                                           needs_layout_passes=False),
  )


def _k3_body(acc_ref, r_ref, b1_ref, w2_ref, as2_ref, ad2_ref,
             t2s_ref, t2d_ref):
  a = acc_ref[0] + acc_ref[1]
  msg = a[:, 0:64]
  den = jnp.dot(a[:, 64:72], r_ref[...], preferred_element_type=jnp.float32)
  z = msg / (den + 1e-16) + b1_ref[0, :]
  z = jnp.where(z > 0, z, jnp.exp(jnp.minimum(z, 0.0)) - 1.0)
  h2 = jnp.dot(z, w2_ref[...], preferred_element_type=jnp.float32)
  asrc = jnp.dot(h2, as2_ref[...], preferred_element_type=jnp.float32)
  adst = jnp.dot(h2, ad2_ref[...], preferred_element_type=jnp.float32)
  z7 = jnp.zeros((h2.shape[0], 7), jnp.float32)
  z8 = jnp.zeros((h2.shape[0], 8), jnp.float32)
  t2s_ref[...] = jnp.concatenate([h2, asrc, z7], axis=1)
  t2d_ref[...] = jnp.concatenate([z8, adst, z7], axis=1)


def _k3(acc1, R, b1, W2, as2, ad2):
  br = 1000
  grid = _N // br
  return pl.pallas_call(
      _k3_body,
      grid=(grid,),
      in_specs=[
          pl.BlockSpec((2, br, 80), lambda i: (0, i, 0)),
          pl.BlockSpec((8, 64), lambda i: (0, 0)),
          pl.BlockSpec((1, 64), lambda i: (0, 0)),
          pl.BlockSpec((64, 8), lambda i: (0, 0)),
          pl.BlockSpec((8, 1), lambda i: (0, 0)),
          pl.BlockSpec((8, 1), lambda i: (0, 0)),
      ],
      out_specs=[
          pl.BlockSpec((br, 16), lambda i: (i, 0)),
          pl.BlockSpec((br, 16), lambda i: (i, 0)),
      ],
      out_shape=[
          jax.ShapeDtypeStruct((_N, 16), jnp.float32),
          jax.ShapeDtypeStruct((_N, 16), jnp.float32),
      ],
  )(acc1, R, b1, W2, as2, ad2)


def _k5_body(acc_ref, batch_ref, p8_ref, b2_ref, f1w_ref, f1b_ref,
             f2w_ref, f2b_ref, o_ref):
  a = acc_ref[0] + acc_ref[1]
  msg = a[0:_N, 0:8]
  den = jnp.dot(a[0:_N, 8:16], p8_ref[...], preferred_element_type=jnp.float32)
  h = msg / (den + 1e-16) + b2_ref[0, :]
  gid = lax.broadcasted_iota(jnp.int32, (_N, _NG), 1)
  bm = (batch_ref[...] == gid).astype(jnp.float32)
  dn = (((0,), (0,)), ((), ()))
  ssum = lax.dot_general(bm, h, dn, preferred_element_type=jnp.float32)
  cnt = lax.dot_general(bm, jnp.ones((_N, 1), jnp.float32), dn,
                        preferred_element_type=jnp.float32)
  g = ssum / jnp.maximum(cnt, 1.0)
  g = jnp.maximum(
      jnp.dot(g, f1w_ref[...], preferred_element_type=jnp.float32)
      + f1b_ref[0, :], 0.0)
  g = jnp.dot(g, f2w_ref[...], preferred_element_type=jnp.float32) \
      + f2b_ref[0, :]
  m = jnp.max(g, axis=1, keepdims=True)
  o_ref[...] = g - (jnp.log(jnp.sum(jnp.exp(g - m), axis=1, keepdims=True))
                    + m)


def _k5(acc2, batch2, P8, b2, fc1_w, fc1_b, fc2_w, fc2_b):
  return pl.pallas_call(
      _k5_body,
      in_specs=[
          pl.BlockSpec((2, _NACC, 16), lambda: (0, 0, 0)),
          pl.BlockSpec((_N, 1), lambda: (0, 0)),
          pl.BlockSpec((8, 8), lambda: (0, 0)),
          pl.BlockSpec((1, 8), lambda: (0, 0)),
          pl.BlockSpec((8, 20), lambda: (0, 0)),
          pl.BlockSpec((1, 20), lambda: (0, 0)),
          pl.BlockSpec((20, 10), lambda: (0, 0)),
          pl.BlockSpec((1, 10), lambda: (0, 0)),
      ],
      out_specs=pl.BlockSpec((_NG, _NC), lambda: (0, 0)),
      out_shape=jax.ShapeDtypeStruct((_NG, _NC), jnp.float32),
  )(acc2, batch2, P8, b2, fc1_w, fc1_b, fc2_w, fc2_b)


def kernel(x, edge_index, batch, W1, a_src1, a_dst1, b1, W2, a_src2, a_dst2,
           b2, fc1_w, fc1_b, fc2_w, fc2_b):
  e = edge_index.shape[1]
  ea = e + _N
  nchunks = -(-ea // (_NW * _CH))
  nchunks += nchunks % 2
  epad = _NW * nchunks * _CH

  sl = jnp.arange(_N, dtype=jnp.int32)
  padi = _N + (jnp.arange(epad - ea, dtype=jnp.int32) % (_NACC - _N))
  srcp = jnp.concatenate([edge_index[0], sl, padi])
  dstp = jnp.concatenate([edge_index[1], sl, padi])
  idxp = jnp.stack([srcp.reshape(_NW, nchunks, _CH),
                    dstp.reshape(_NW, nchunks, _CH)], axis=2)

  eye8 = jnp.eye(8, dtype=jnp.float32)
  W1dm = W1.reshape(128, 8, 8).transpose(0, 2, 1).reshape(128, 64)
  b1dm = b1.reshape(8, 8).transpose(1, 0).reshape(1, 64)
  W2dm = W2.reshape(8, 8, 8).transpose(1, 0, 2).reshape(64, 8)
  A1s = (a_src1.T[:, :, None] * eye8[None, :, :]).reshape(64, 8)
  A1d = (a_dst1.T[:, :, None] * eye8[None, :, :]).reshape(64, 8)
  R = jnp.tile(eye8, (1, 8))
  P8 = jnp.zeros((8, 8), jnp.float32).at[0, :].set(1.0)
  as2 = a_src2.reshape(8, 1)
  ad2 = a_dst2.reshape(8, 1)

  t1, d1 = _k1(x, W1dm, A1s, A1d)
  t1p = jnp.zeros((_NACC, 80), jnp.float32).at[:_N].set(t1)
  d1p = jnp.zeros((_NACC, 16), jnp.float32).at[:_N].set(d1)

  acc1 = _edge_pass(80, nchunks)(t1p, d1p, idxp)

  t2s, t2d = _k3(acc1, R, b1dm, W2dm, as2, ad2)
  t2sp = jnp.zeros((_NACC, 16), jnp.float32).at[:_N].set(t2s)
  t2dp = jnp.zeros((_NACC, 16), jnp.float32).at[:_N].set(t2d)

  acc2 = _edge_pass(16, nchunks)(t2sp, t2dp, idxp)

  return _k5(acc2, batch.reshape(_N, 1), P8, b2.reshape(1, 8),
             fc1_w, fc1_b.reshape(1, 20), fc2_w, fc2_b.reshape(1, 10))


# speedup vs baseline: 1.0123x; 1.0123x over previous
import functools

import jax
import jax.numpy as jnp
from jax import lax
from jax.experimental import pallas as pl
from jax.experimental.pallas import tpu as pltpu
from jax.experimental.pallas import tpu_sc as plsc

_N = 10000
_NG = 64
_NC = 10
_NW = 32
_CH = 128
_NACC = 10240
_RPS = _NACC // 16


def _leaky(z):
  return jnp.maximum(z, 0.0) + 0.2 * jnp.minimum(z, 0.0)


def _splat8(vec, eq8):
  return jnp.full((16,), jnp.sum(jnp.where(eq8, vec, 0.0), axis=0))


def _k1_body(x_ref, w_ref, as_ref, ad_ref, t1_ref, d1_ref):
  h = jnp.dot(x_ref[...], w_ref[...], preferred_element_type=jnp.float32)
  asrc = jnp.dot(h, as_ref[...], preferred_element_type=jnp.float32)
  adst = jnp.dot(h, ad_ref[...], preferred_element_type=jnp.float32)
  t1_ref[...] = jnp.concatenate([h, asrc, asrc], axis=1)
  d1_ref[...] = jnp.concatenate([adst, adst], axis=1)


def _k1(x, W1, A1s, A1d):
  br = 1000
  grid = _N // br
  return pl.pallas_call(
      _k1_body,
      grid=(grid,),
      in_specs=[
          pl.BlockSpec((br, 128), lambda i: (i, 0)),
          pl.BlockSpec((128, 64), lambda i: (0, 0)),
          pl.BlockSpec((64, 8), lambda i: (0, 0)),
          pl.BlockSpec((64, 8), lambda i: (0, 0)),
      ],
      out_specs=[
          pl.BlockSpec((br, 80), lambda i: (i, 0)),
          pl.BlockSpec((br, 16), lambda i: (i, 0)),
      ],
      out_shape=[
          jax.ShapeDtypeStruct((_N, 80), jnp.float32),
          jax.ShapeDtypeStruct((_N, 16), jnp.float32),
      ],
  )(x, W1, A1s, A1d)


def _edge_pass(width, nchunks):
  mesh = plsc.VectorSubcoreMesh(core_axis_name="c", subcore_axis_name="s")
  assert nchunks % 2 == 0
  k2 = nchunks // 2

  def body(t_hbm, d_hbm, idx_hbm, acc_hbm,
           idxall, sbufA, sbufB, dbufA, dbufB, mbufA, mbufB, accs,
           gsemA, gsemB, ssemA, ssemB):
    c = lax.axis_index("c")
    s = lax.axis_index("s")
    wid = c * 16 + s
    lane = lax.iota(jnp.int32, 16)
    lt8 = lane < 8
    eq8 = lane == 8
    zero16 = (lane * 0).astype(jnp.float32)

    pltpu.sync_copy(idx_hbm.at[wid], idxall)

    def zrow(r, _):
      for j in range(width // 16):
        mbufA[r, j * 16:(j + 1) * 16] = zero16
      return 0
    lax.fori_loop(0, _CH, zrow, 0)
    for t in range(_RPS // _CH):
      pltpu.sync_copy(mbufA, accs.at[pl.ds(s * _RPS + t * _CH, _CH)])
    plsc.subcore_barrier()

    def gathers(k, sbuf, dbuf, gsem):
      pltpu.async_copy(t_hbm.at[idxall.at[k, 0]], sbuf, gsem)
      pltpu.async_copy(d_hbm.at[idxall.at[k, 1]], dbuf, gsem)

    def wait_gathers(k, sbuf, dbuf, gsem):
      pltpu.make_async_copy(t_hbm.at[idxall.at[k, 0]], sbuf, gsem).wait()
      pltpu.make_async_copy(d_hbm.at[idxall.at[k, 1]], dbuf, gsem).wait()

    def wait_scatter(k, mbuf, ssem):
      pltpu.make_async_copy(mbuf, accs.at[idxall.at[k, 1]], ssem).wait()

    if width == 80:
      def one_row(r, sbuf, dbuf, mbuf):
        asr = sbuf[r, 64:80]
        ads = dbuf[r, 0:16]
        e16 = jnp.exp(_leaky(asr + ads))
        for j in range(4):
          mbuf[r, j * 16:(j + 1) * 16] = e16 * sbuf[r, j * 16:(j + 1) * 16]
        mbuf[r, 64:80] = jnp.where(lt8, e16, 0.0)
    else:
      def one_row(r, sbuf, dbuf, mbuf):
        v = sbuf[r, 0:16]
        w = dbuf[r, 0:16]
        e16 = jnp.exp(_leaky(_splat8(v + w, eq8)))
        mbuf[r, 0:16] = jnp.where(lt8, e16 * v, jnp.where(eq8, e16, 0.0))

    def compute(sbuf, dbuf, mbuf):
      @plsc.parallel_loop(0, _CH, 1, unroll=8)
      def _rows(r):
        one_row(r, sbuf, dbuf, mbuf)

    gathers(0, sbufA, dbufA, gsemA)

    def pair(i, _):
      ka = 2 * i
      kb = 2 * i + 1
      gathers(kb, sbufB, dbufB, gsemB)
      wait_gathers(ka, sbufA, dbufA, gsemA)
      pl.when(i > 0)(lambda: wait_scatter(ka, mbufA, ssemA))
      compute(sbufA, dbufA, mbufA)
      pltpu.async_copy(mbufA, accs.at[idxall.at[ka, 1]], ssemA, add=True)
      pl.when(i < k2 - 1)(lambda: gathers(ka + 2, sbufA, dbufA, gsemA))
      wait_gathers(kb, sbufB, dbufB, gsemB)
      pl.when(i > 0)(lambda: wait_scatter(kb, mbufB, ssemB))
      compute(sbufB, dbufB, mbufB)
      pltpu.async_copy(mbufB, accs.at[idxall.at[kb, 1]], ssemB, add=True)
      return 0

    lax.fori_loop(0, k2, pair, 0)
    wait_scatter(nchunks - 2, mbufA, ssemA)
    wait_scatter(nchunks - 1, mbufB, ssemB)
    plsc.subcore_barrier()
    for t in range(_RPS // _CH):
      r0 = s * _RPS + t * _CH
      pltpu.sync_copy(accs.at[pl.ds(r0, _CH)], acc_hbm.at[c, pl.ds(r0, _CH)])

  return pl.kernel(
      body,
      out_type=jax.ShapeDtypeStruct((2, _NACC, width), jnp.float32),
      mesh=mesh,
      scratch_types=[
          pltpu.VMEM((nchunks, 2, _CH), jnp.int32),
          pltpu.VMEM((_CH, width), jnp.float32),
          pltpu.VMEM((_CH, width), jnp.float32),
          pltpu.VMEM((_CH, 16), jnp.float32),
          pltpu.VMEM((_CH, 16), jnp.float32),
          pltpu.VMEM((_CH, width), jnp.float32),
          pltpu.VMEM((_CH, width), jnp.float32),
          pltpu.VMEM_SHARED((_NACC, width), jnp.float32),
          pltpu.SemaphoreType.DMA,
          pltpu.SemaphoreType.DMA,
          pltpu.SemaphoreType.DMA,
          pltpu.SemaphoreType.DMA,
      ],
      compiler_params=pltpu.CompilerParams(use_tc_tiling_on_sc=False,
                                           needs_layout_passes=False),
  )


def _k3_body(acc_ref, r_ref, b1_ref, w2_ref, as2_ref, ad2_ref,
             t2s_ref, t2d_ref):
  a = acc_ref[0] + acc_ref[1]
  msg = a[:, 0:64]
  den = jnp.dot(a[:, 64:72], r_ref[...], preferred_element_type=jnp.float32)
  z = msg / (den + 1e-16) + b1_ref[0, :]
  z = jnp.where(z > 0, z, jnp.exp(jnp.minimum(z, 0.0)) - 1.0)
  h2 = jnp.dot(z, w2_ref[...], preferred_element_type=jnp.float32)
  asrc = jnp.dot(h2, as2_ref[...], preferred_element_type=jnp.float32)
  adst = jnp.dot(h2, ad2_ref[...], preferred_element_type=jnp.float32)
  z7 = jnp.zeros((h2.shape[0], 7), jnp.float32)
  z8 = jnp.zeros((h2.shape[0], 8), jnp.float32)
  t2s_ref[...] = jnp.concatenate([h2, asrc, z7], axis=1)
  t2d_ref[...] = jnp.concatenate([z8, adst, z7], axis=1)


def _k3(acc1, R, b1, W2, as2, ad2):
  br = 1000
  grid = _N // br
  return pl.pallas_call(
      _k3_body,
      grid=(grid,),
      in_specs=[
          pl.BlockSpec((2, br, 80), lambda i: (0, i, 0)),
          pl.BlockSpec((8, 64), lambda i: (0, 0)),
          pl.BlockSpec((1, 64), lambda i: (0, 0)),
          pl.BlockSpec((64, 8), lambda i: (0, 0)),
          pl.BlockSpec((8, 1), lambda i: (0, 0)),
          pl.BlockSpec((8, 1), lambda i: (0, 0)),
      ],
      out_specs=[
          pl.BlockSpec((br, 16), lambda i: (i, 0)),
          pl.BlockSpec((br, 16), lambda i: (i, 0)),
      ],
      out_shape=[
          jax.ShapeDtypeStruct((_N, 16), jnp.float32),
          jax.ShapeDtypeStruct((_N, 16), jnp.float32),
      ],
  )(acc1, R, b1, W2, as2, ad2)


def _k5_body(acc_ref, batch_ref, p8_ref, b2_ref, f1w_ref, f1b_ref,
             f2w_ref, f2b_ref, o_ref):
  a = acc_ref[0] + acc_ref[1]
  msg = a[0:_N, 0:8]
  den = jnp.dot(a[0:_N, 8:16], p8_ref[...], preferred_element_type=jnp.float32)
  h = msg / (den + 1e-16) + b2_ref[0, :]
  gid = lax.broadcasted_iota(jnp.int32, (_N, _NG), 1)
  bm = (batch_ref[...] == gid).astype(jnp.float32)
  dn = (((0,), (0,)), ((), ()))
  ssum = lax.dot_general(bm, h, dn, preferred_element_type=jnp.float32)
  cnt = lax.dot_general(bm, jnp.ones((_N, 1), jnp.float32), dn,
                        preferred_element_type=jnp.float32)
  g = ssum / jnp.maximum(cnt, 1.0)
  g = jnp.maximum(
      jnp.dot(g, f1w_ref[...], preferred_element_type=jnp.float32)
      + f1b_ref[0, :], 0.0)
  g = jnp.dot(g, f2w_ref[...], preferred_element_type=jnp.float32) \
      + f2b_ref[0, :]
  m = jnp.max(g, axis=1, keepdims=True)
  o_ref[...] = g - (jnp.log(jnp.sum(jnp.exp(g - m), axis=1, keepdims=True))
                    + m)


def _k5(acc2, batch2, P8, b2, fc1_w, fc1_b, fc2_w, fc2_b):
  return pl.pallas_call(
      _k5_body,
      in_specs=[
          pl.BlockSpec((2, _NACC, 16), lambda: (0, 0, 0)),
          pl.BlockSpec((_N, 1), lambda: (0, 0)),
          pl.BlockSpec((8, 8), lambda: (0, 0)),
          pl.BlockSpec((1, 8), lambda: (0, 0)),
          pl.BlockSpec((8, 20), lambda: (0, 0)),
          pl.BlockSpec((1, 20), lambda: (0, 0)),
          pl.BlockSpec((20, 10), lambda: (0, 0)),
          pl.BlockSpec((1, 10), lambda: (0, 0)),
      ],
      out_specs=pl.BlockSpec((_NG, _NC), lambda: (0, 0)),
      out_shape=jax.ShapeDtypeStruct((_NG, _NC), jnp.float32),
  )(acc2, batch2, P8, b2, fc1_w, fc1_b, fc2_w, fc2_b)


def kernel(x, edge_index, batch, W1, a_src1, a_dst1, b1, W2, a_src2, a_dst2,
           b2, fc1_w, fc1_b, fc2_w, fc2_b):
  e = edge_index.shape[1]
  ea = e + _N
  nchunks = -(-ea // (_NW * _CH))
  nchunks += nchunks % 2
  epad = _NW * nchunks * _CH

  sl = jnp.arange(_N, dtype=jnp.int32)
  padi = _N + (jnp.arange(epad - ea, dtype=jnp.int32) % (_NACC - _N))
  srcp = jnp.concatenate([edge_index[0], sl, padi])
  dstp = jnp.concatenate([edge_index[1], sl, padi])
  idxp = jnp.stack([srcp.reshape(_NW, nchunks, _CH),
                    dstp.reshape(_NW, nchunks, _CH)], axis=2)

  eye8 = jnp.eye(8, dtype=jnp.float32)
  W1dm = W1.reshape(128, 8, 8).transpose(0, 2, 1).reshape(128, 64)
  b1dm = b1.reshape(8, 8).transpose(1, 0).reshape(1, 64)
  W2dm = W2.reshape(8, 8, 8).transpose(1, 0, 2).reshape(64, 8)
  A1s = (a_src1.T[:, :, None] * eye8[None, :, :]).reshape(64, 8)
  A1d = (a_dst1.T[:, :, None] * eye8[None, :, :]).reshape(64, 8)
  R = jnp.tile(eye8, (1, 8))
  P8 = jnp.zeros((8, 8), jnp.float32).at[0, :].set(1.0)
  as2 = a_src2.reshape(8, 1)
  ad2 = a_dst2.reshape(8, 1)

  t1, d1 = _k1(x, W1dm, A1s, A1d)
  t1p = jnp.zeros((_NACC, 80), jnp.float32).at[:_N].set(t1)
  d1p = jnp.zeros((_NACC, 16), jnp.float32).at[:_N].set(d1)

  acc1 = _edge_pass(80, nchunks)(t1p, d1p, idxp)

  t2s, t2d = _k3(acc1, R, b1dm, W2dm, as2, ad2)
  t2sp = jnp.zeros((_NACC, 16), jnp.float32).at[:_N].set(t2s)
  t2dp = jnp.zeros((_NACC, 16), jnp.float32).at[:_N].set(t2d)

  acc2 = _edge_pass(16, nchunks)(t2sp, t2dp, idxp)

  return _k5(acc2, batch.reshape(_N, 1), P8, b2.reshape(1, 8),
             fc1_w, fc1_b.reshape(1, 20), fc2_w, fc2_b.reshape(1, 10))


# speedup vs baseline: 1.0129x; 1.0006x over previous
import jax
import jax.numpy as jnp
from jax import lax
from jax.experimental import pallas as pl
from jax.experimental.pallas import tpu as pltpu
from jax.experimental.pallas import tpu_sc as plsc

_N = 10000
_NG = 64
_NC = 10
_NW = 32
_CH = 128
_NACC = 10240
_RPS = _NACC // 16


def _leaky(z):
  return jnp.maximum(z, 0.0) + 0.2 * jnp.minimum(z, 0.0)


def _splat8(vec, eq8):
  return jnp.full((16,), jnp.sum(jnp.where(eq8, vec, 0.0), axis=0))


def _k1_body(x_ref, w_ref, as_ref, ad_ref, t1_ref, d1_ref):
  h = jnp.dot(x_ref[...], w_ref[...], preferred_element_type=jnp.float32)
  asrc = jnp.dot(h, as_ref[...], preferred_element_type=jnp.float32)
  adst = jnp.dot(h, ad_ref[...], preferred_element_type=jnp.float32)
  t1_ref[...] = jnp.concatenate([h, asrc, asrc], axis=1)
  d1_ref[...] = jnp.concatenate([adst, adst], axis=1)


def _k1(x, W1, A1s, A1d):
  br = 1000
  grid = _N // br
  return pl.pallas_call(
      _k1_body,
      grid=(grid,),
      in_specs=[
          pl.BlockSpec((br, 128), lambda i: (i, 0)),
          pl.BlockSpec((128, 64), lambda i: (0, 0)),
          pl.BlockSpec((64, 8), lambda i: (0, 0)),
          pl.BlockSpec((64, 8), lambda i: (0, 0)),
      ],
      out_specs=[
          pl.BlockSpec((br, 80), lambda i: (i, 0)),
          pl.BlockSpec((br, 16), lambda i: (i, 0)),
      ],
      out_shape=[
          jax.ShapeDtypeStruct((_N, 80), jnp.float32),
          jax.ShapeDtypeStruct((_N, 16), jnp.float32),
      ],
  )(x, W1, A1s, A1d)


def _edge_pass(width, nchunks):
  mesh = plsc.VectorSubcoreMesh(core_axis_name="c", subcore_axis_name="s")
  assert nchunks % 2 == 0
  k2 = nchunks // 2

  def body(t_hbm, d_hbm, idx_hbm, acc_hbm,
           idxall, sbufA, sbufB, dbufA, dbufB, mbufA, mbufB, accs,
           gsemA, gsemB, ssemA, ssemB):
    c = lax.axis_index("c")
    s = lax.axis_index("s")
    wid = c * 16 + s
    lane = lax.iota(jnp.int32, 16)
    lt8 = lane < 8
    eq8 = lane == 8
    zero16 = (lane * 0).astype(jnp.float32)

    pltpu.sync_copy(idx_hbm.at[wid], idxall)

    def zrow(r, _):
      for j in range(width // 16):
        mbufA[r, j * 16:(j + 1) * 16] = zero16
      return 0
    lax.fori_loop(0, _CH, zrow, 0)
    for t in range(_RPS // _CH):
      pltpu.sync_copy(mbufA, accs.at[pl.ds(s * _RPS + t * _CH, _CH)])
    plsc.subcore_barrier()

    def gathers(k, sbuf, dbuf, gsem):
      pltpu.async_copy(t_hbm.at[idxall.at[k, 0]], sbuf, gsem)
      pltpu.async_copy(d_hbm.at[idxall.at[k, 1]], dbuf, gsem)

    def wait_gathers(k, sbuf, dbuf, gsem):
      pltpu.make_async_copy(t_hbm.at[idxall.at[k, 0]], sbuf, gsem).wait()
      pltpu.make_async_copy(d_hbm.at[idxall.at[k, 1]], dbuf, gsem).wait()

    def wait_scatter(k, mbuf, ssem):
      pltpu.make_async_copy(mbuf, accs.at[idxall.at[k, 1]], ssem).wait()

    if width == 80:
      def one_row(r, sbuf, dbuf, mbuf):
        asr = sbuf[r, 64:80]
        ads = dbuf[r, 0:16]
        e16 = jnp.exp(_leaky(asr + ads))
        for j in range(4):
          mbuf[r, j * 16:(j + 1) * 16] = e16 * sbuf[r, j * 16:(j + 1) * 16]
        mbuf[r, 64:80] = jnp.where(lt8, e16, 0.0)
    else:
      def one_row(r, sbuf, dbuf, mbuf):
        v = sbuf[r, 0:16]
        w = dbuf[r, 0:16]
        e16 = jnp.exp(_leaky(_splat8(v + w, eq8)))
        mbuf[r, 0:16] = jnp.where(lt8, e16 * v, jnp.where(eq8, e16, 0.0))

    def compute(sbuf, dbuf, mbuf):
      @plsc.parallel_loop(0, _CH, 1, unroll=8)
      def _rows(r):
        one_row(r, sbuf, dbuf, mbuf)

    gathers(0, sbufA, dbufA, gsemA)

    def pair(i, _):
      ka = 2 * i
      kb = 2 * i + 1
      gathers(kb, sbufB, dbufB, gsemB)
      wait_gathers(ka, sbufA, dbufA, gsemA)
      pl.when(i > 0)(lambda: wait_scatter(ka, mbufA, ssemA))
      compute(sbufA, dbufA, mbufA)
      pltpu.async_copy(mbufA, accs.at[idxall.at[ka, 1]], ssemA, add=True)
      pl.when(i < k2 - 1)(lambda: gathers(ka + 2, sbufA, dbufA, gsemA))
      wait_gathers(kb, sbufB, dbufB, gsemB)
      pl.when(i > 0)(lambda: wait_scatter(kb, mbufB, ssemB))
      compute(sbufB, dbufB, mbufB)
      pltpu.async_copy(mbufB, accs.at[idxall.at[kb, 1]], ssemB, add=True)
      return 0

    lax.fori_loop(0, k2, pair, 0)
    wait_scatter(nchunks - 2, mbufA, ssemA)
    wait_scatter(nchunks - 1, mbufB, ssemB)
    plsc.subcore_barrier()
    for t in range(_RPS // _CH):
      r0 = s * _RPS + t * _CH
      pltpu.sync_copy(accs.at[pl.ds(r0, _CH)], acc_hbm.at[c, pl.ds(r0, _CH)])

  return pl.kernel(
      body,
      out_type=jax.ShapeDtypeStruct((2, _NACC, width), jnp.float32),
      mesh=mesh,
      scratch_types=[
          pltpu.VMEM((nchunks, 2, _CH), jnp.int32),
          pltpu.VMEM((_CH, width), jnp.float32),
          pltpu.VMEM((_CH, width), jnp.float32),
          pltpu.VMEM((_CH, 16), jnp.float32),
          pltpu.VMEM((_CH, 16), jnp.float32),
          pltpu.VMEM((_CH, width), jnp.float32),
          pltpu.VMEM((_CH, width), jnp.float32),
          pltpu.VMEM_SHARED((_NACC, width), jnp.float32),
          pltpu.SemaphoreType.DMA,
          pltpu.SemaphoreType.DMA,
          pltpu.SemaphoreType.DMA,
          pltpu.SemaphoreType.DMA,
      ],
      compiler_params=pltpu.CompilerParams(use_tc_tiling_on_sc=False,
                                           needs_layout_passes=False),
  )


def _k3_body(acc_ref, r_ref, b1_ref, w2_ref, as2_ref, ad2_ref,
             t2s_ref, t2d_ref):
  a = acc_ref[0] + acc_ref[1]
  msg = a[:, 0:64]
  den = jnp.dot(a[:, 64:72], r_ref[...], preferred_element_type=jnp.float32)
  z = msg / (den + 1e-16) + b1_ref[0, :]
  z = jnp.where(z > 0, z, jnp.exp(jnp.minimum(z, 0.0)) - 1.0)
  h2 = jnp.dot(z, w2_ref[...], preferred_element_type=jnp.float32)
  asrc = jnp.dot(h2, as2_ref[...], preferred_element_type=jnp.float32)
  adst = jnp.dot(h2, ad2_ref[...], preferred_element_type=jnp.float32)
  z7 = jnp.zeros((h2.shape[0], 7), jnp.float32)
  z8 = jnp.zeros((h2.shape[0], 8), jnp.float32)
  t2s_ref[...] = jnp.concatenate([h2, asrc, z7], axis=1)
  t2d_ref[...] = jnp.concatenate([z8, adst, z7], axis=1)


def _k3(acc1, R, b1, W2, as2, ad2):
  br = 1000
  grid = _N // br
  return pl.pallas_call(
      _k3_body,
      grid=(grid,),
      in_specs=[
          pl.BlockSpec((2, br, 80), lambda i: (0, i, 0)),
          pl.BlockSpec((8, 64), lambda i: (0, 0)),
          pl.BlockSpec((1, 64), lambda i: (0, 0)),
          pl.BlockSpec((64, 8), lambda i: (0, 0)),
          pl.BlockSpec((8, 1), lambda i: (0, 0)),
          pl.BlockSpec((8, 1), lambda i: (0, 0)),
      ],
      out_specs=[
          pl.BlockSpec((br, 16), lambda i: (i, 0)),
          pl.BlockSpec((br, 16), lambda i: (i, 0)),
      ],
      out_shape=[
          jax.ShapeDtypeStruct((_N, 16), jnp.float32),
          jax.ShapeDtypeStruct((_N, 16), jnp.float32),
      ],
  )(acc1, R, b1, W2, as2, ad2)


def _k5_body(acc_ref, batch_ref, p8_ref, b2_ref, f1w_ref, f1b_ref,
             f2w_ref, f2b_ref, o_ref):
  a = acc_ref[0] + acc_ref[1]
  msg = a[0:_N, 0:8]
  den = jnp.dot(a[0:_N, 8:16], p8_ref[...], preferred_element_type=jnp.float32)
  h = msg / (den + 1e-16) + b2_ref[0, :]
  gid = lax.broadcasted_iota(jnp.int32, (_N, _NG), 1)
  bm = (batch_ref[...] == gid).astype(jnp.float32)
  dn = (((0,), (0,)), ((), ()))
  ssum = lax.dot_general(bm, h, dn, preferred_element_type=jnp.float32)
  cnt = lax.dot_general(bm, jnp.ones((_N, 1), jnp.float32), dn,
                        preferred_element_type=jnp.float32)
  g = ssum / jnp.maximum(cnt, 1.0)
  g = jnp.maximum(
      jnp.dot(g, f1w_ref[...], preferred_element_type=jnp.float32)
      + f1b_ref[0, :], 0.0)
  g = jnp.dot(g, f2w_ref[...], preferred_element_type=jnp.float32) \
      + f2b_ref[0, :]
  m = jnp.max(g, axis=1, keepdims=True)
  o_ref[...] = g - (jnp.log(jnp.sum(jnp.exp(g - m), axis=1, keepdims=True))
                    + m)


def _k5(acc2, batch2, P8, b2, fc1_w, fc1_b, fc2_w, fc2_b):
  return pl.pallas_call(
      _k5_body,
      in_specs=[
          pl.BlockSpec((2, _NACC, 16), lambda: (0, 0, 0)),
          pl.BlockSpec((_N, 1), lambda: (0, 0)),
          pl.BlockSpec((8, 8), lambda: (0, 0)),
          pl.BlockSpec((1, 8), lambda: (0, 0)),
          pl.BlockSpec((8, 20), lambda: (0, 0)),
          pl.BlockSpec((1, 20), lambda: (0, 0)),
          pl.BlockSpec((20, 10), lambda: (0, 0)),
          pl.BlockSpec((1, 10), lambda: (0, 0)),
      ],
      out_specs=pl.BlockSpec((_NG, _NC), lambda: (0, 0)),
      out_shape=jax.ShapeDtypeStruct((_NG, _NC), jnp.float32),
  )(acc2, batch2, P8, b2, fc1_w, fc1_b, fc2_w, fc2_b)


def kernel(x, edge_index, batch, W1, a_src1, a_dst1, b1, W2, a_src2, a_dst2,
           b2, fc1_w, fc1_b, fc2_w, fc2_b):
  e = edge_index.shape[1]
  ea = e + _N
  nchunks = -(-ea // (_NW * _CH))
  nchunks += nchunks % 2
  epad = _NW * nchunks * _CH

  sl = jnp.arange(_N, dtype=jnp.int32)
  padi = _N + (jnp.arange(epad - ea, dtype=jnp.int32) % (_NACC - _N))
  srcp = jnp.concatenate([edge_index[0], sl, padi])
  dstp = jnp.concatenate([edge_index[1], sl, padi])
  idxp = jnp.stack([srcp.reshape(_NW, nchunks, _CH),
                    dstp.reshape(_NW, nchunks, _CH)], axis=2)

  eye8 = jnp.eye(8, dtype=jnp.float32)
  W1dm = W1.reshape(128, 8, 8).transpose(0, 2, 1).reshape(128, 64)
  b1dm = b1.reshape(8, 8).transpose(1, 0).reshape(1, 64)
  W2dm = W2.reshape(8, 8, 8).transpose(1, 0, 2).reshape(64, 8)
  A1s = (a_src1.T[:, :, None] * eye8[None, :, :]).reshape(64, 8)
  A1d = (a_dst1.T[:, :, None] * eye8[None, :, :]).reshape(64, 8)
  R = jnp.tile(eye8, (1, 8))
  P8 = jnp.zeros((8, 8), jnp.float32).at[0, :].set(1.0)
  as2 = a_src2.reshape(8, 1)
  ad2 = a_dst2.reshape(8, 1)

  t1, d1 = _k1(x, W1dm, A1s, A1d)
  t1p = jnp.zeros((_NACC, 80), jnp.float32).at[:_N].set(t1)
  d1p = jnp.zeros((_NACC, 16), jnp.float32).at[:_N].set(d1)

  acc1 = _edge_pass(80, nchunks)(t1p, d1p, idxp)

  t2s, t2d = _k3(acc1, R, b1dm, W2dm, as2, ad2)
  t2sp = jnp.zeros((_NACC, 16), jnp.float32).at[:_N].set(t2s)
  t2dp = jnp.zeros((_NACC, 16), jnp.float32).at[:_N].set(t2d)

  acc2 = _edge_pass(16, nchunks)(t2sp, t2dp, idxp)

  return _k5(acc2, batch.reshape(_N, 1), P8, b2.reshape(1, 8),
             fc1_w, fc1_b.reshape(1, 20), fc2_w, fc2_b.reshape(1, 10))
